# Initial kernel scaffold; baseline (speedup 1.0000x reference)
import jax
import jax.numpy as jnp
from jax.experimental import pallas as pl


def kernel(x, edge_index, W1, b1, W2, b2, W3, b3):
    raise NotImplementedError("write your pallas kernel here")


# speedup vs baseline: 10.3447x; 10.3447x over previous
import functools

import jax
import jax.numpy as jnp
from jax import lax
from jax.experimental import pallas as pl
from jax.experimental.pallas import tpu as pltpu
from jax.experimental.pallas import tpu_sc as plsc

N = 50000
E = 1_600_000
IN_DIM, H1, H2 = 48, 300, 100

NP = 50176
STRIPE = NP // 32
PE = 1_638_400
NB = PE // 128
NC, NS = 2, 16
PERW = NB // (NC * NS)
GRP = 8
NGRP = PERW // GRP

CH = 16
BLK = 512
NBLK = NP // BLK


def _mesh():
    return plsc.VectorSubcoreMesh(core_axis_name="c", subcore_axis_name="s")


def _sc_agg_rows(tables, src2d, dst2d, zeros):
    nt = len(tables)

    def body(*refs):
        t_refs = refs[:nt]
        src_r, dst_r, z_r = refs[nt], refs[nt + 1], refs[nt + 2]
        out_refs = refs[nt + 3:nt + 3 + nt]
        acc, src_v, dst_v, rows_v, zv = refs[nt + 3 + nt:]
        c = lax.axis_index("c")
        s = lax.axis_index("s")
        w = s * NC + c
        base_row = w * PERW
        for ti in range(nt):
            pltpu.sync_copy(z_r, zv)
            pltpu.sync_copy(zv, acc.at[pl.ds(s * STRIPE, STRIPE)])
            plsc.subcore_barrier()

            def grp(g, _):
                row0 = base_row + g * GRP
                pltpu.sync_copy(src_r.at[pl.ds(row0, GRP)], src_v)
                pltpu.sync_copy(dst_r.at[pl.ds(row0, GRP)], dst_v)
                for j in range(GRP):
                    pltpu.sync_copy(t_refs[ti].at[src_v.at[j]], rows_v)
                    pltpu.sync_copy(rows_v, acc.at[dst_v.at[j]], add=True)
                return 0

            lax.fori_loop(0, NGRP, grp, 0)
            plsc.subcore_barrier()
            pltpu.sync_copy(acc.at[pl.ds(s * STRIPE, STRIPE)], zv)
            pltpu.sync_copy(zv,
                            out_refs[ti].at[pl.ds(c * NP + s * STRIPE, STRIPE)])
            plsc.subcore_barrier()

    out_t = [jax.ShapeDtypeStruct((2 * NP, CH), jnp.float32) for _ in range(nt)]
    scratch = [
        pltpu.VMEM_SHARED((NP, CH), jnp.float32),
        pltpu.VMEM((GRP, 128), jnp.int32),
        pltpu.VMEM((GRP, 128), jnp.int32),
        pltpu.VMEM((128, CH), jnp.float32),
        pltpu.VMEM((STRIPE, CH), jnp.float32),
    ]
    f = pl.kernel(body, out_type=out_t, mesh=_mesh(), scratch_types=scratch,
                  compiler_params=pltpu.CompilerParams(
                      use_tc_tiling_on_sc=False))
    outs = f(*tables, src2d, dst2d, zeros)
    return list(outs) if nt > 1 else [outs]


def _sc_agg_scalar(table, src2d, dst2d, zeros1):
    def body(t_r, src_r, dst_r, z_r, out_r, acc, tbl_v, src_v, dst_v, vals_v, zv):
        c = lax.axis_index("c")
        s = lax.axis_index("s")
        w = s * NC + c
        base_row = w * PERW
        pltpu.sync_copy(z_r, zv)
        pltpu.sync_copy(zv, acc.at[pl.ds(s * STRIPE, STRIPE)])
        pltpu.sync_copy(t_r, tbl_v)
        plsc.subcore_barrier()

        def grp(g, _):
            row0 = base_row + g * GRP
            pltpu.sync_copy(src_r.at[pl.ds(row0, GRP)], src_v)
            pltpu.sync_copy(dst_r.at[pl.ds(row0, GRP)], dst_v)
            for j in range(GRP):
                for k in range(8):
                    idx16 = src_v[j, pl.ds(k * 16, 16)]
                    vals_v[pl.ds(k * 16, 16)] = plsc.load_gather(tbl_v, [idx16])
                pltpu.sync_copy(vals_v, acc.at[dst_v.at[j]], add=True)
            return 0

        lax.fori_loop(0, NGRP, grp, 0)
        plsc.subcore_barrier()
        pltpu.sync_copy(acc.at[pl.ds(s * STRIPE, STRIPE)], zv)
        pltpu.sync_copy(zv, out_r.at[pl.ds(c * NP + s * STRIPE, STRIPE)])

    scratch = [
        pltpu.VMEM_SHARED((NP,), jnp.float32),
        pltpu.VMEM((NP,), jnp.float32),
        pltpu.VMEM((GRP, 128), jnp.int32),
        pltpu.VMEM((GRP, 128), jnp.int32),
        pltpu.VMEM((128,), jnp.float32),
        pltpu.VMEM((STRIPE,), jnp.float32),
    ]
    f = pl.kernel(body, out_type=jax.ShapeDtypeStruct((2 * NP,), jnp.float32),
                  mesh=_mesh(), scratch_types=scratch,
                  compiler_params=pltpu.CompilerParams(
                      needs_layout_passes=False, use_tc_tiling_on_sc=False))
    return f(table, src2d, dst2d, zeros1)


def _row_mask(pid):
    rows = pid * BLK + lax.broadcasted_iota(jnp.int32, (BLK, 1), 0)
    return (rows < N).astype(jnp.float32)


def _tc1_body(d0, d1, x, dinv_o, x0_o, x1_o, x2_o):
    deg = d0[...] + d1[...] + 1.0
    dinv = lax.rsqrt(deg)
    m = _row_mask(pl.program_id(0))
    xs = (m * dinv) * x[...]
    dinv_o[...] = dinv
    x0_o[...] = xs[:, 0:16]
    x1_o[...] = xs[:, 16:32]
    x2_o[...] = xs[:, 32:48]


def _tc2_body(a00, a01, a10, a11, a20, a21, x, dinv_r, W1, b1r, W2, *outs):
    agg = jnp.concatenate([a00[...] + a01[...], a10[...] + a11[...],
                           a20[...] + a21[...]], axis=1)
    dinv = dinv_r[...]
    pre = dinv * agg + (dinv * dinv) * x[...]
    h1 = jnp.maximum(jnp.dot(pre, W1[...],
                             preferred_element_type=jnp.float32) + b1r[...], 0.0)
    t2 = jnp.dot(h1, W2[...], preferred_element_type=jnp.float32)
    m = _row_mask(pl.program_id(0))
    xs2 = (m * dinv) * t2
    outs[0][...] = t2
    for ci in range(6):
        outs[1 + ci][...] = xs2[:, 16 * ci:16 * ci + 16]
    outs[7][...] = jnp.concatenate(
        [xs2[:, 96:100], jnp.zeros((BLK, 12), jnp.float32)], axis=1)


def _tc3_body(*refs):
    aggs = refs[:14]
    t2, dinv_r, b2r, W3, ts3_o, t3_o = refs[14:]
    agg = jnp.concatenate([aggs[2 * i][...] + aggs[2 * i + 1][...]
                           for i in range(7)], axis=1)[:, :100]
    dinv = dinv_r[...]
    h2 = jnp.maximum(dinv * agg + (dinv * dinv) * t2[...] + b2r[...], 0.0)
    t3 = jnp.dot(h2, W3[...], preferred_element_type=jnp.float32)
    m = _row_mask(pl.program_id(0))
    ts3_o[...] = (m * dinv) * t3
    t3_o[...] = t3


def _tc4_body(g0, g1, t3, dinv_r, b3r, out_o):
    dinv = dinv_r[...]
    out_o[...] = dinv * (g0[...] + g1[...]) + (dinv * dinv) * t3[...] + b3r[...]


def _rowspec(cols):
    return pl.BlockSpec((BLK, cols), lambda i: (i, 0))


def _fullspec(shape):
    return pl.BlockSpec(shape, lambda i: tuple(0 for _ in shape))


def _tc_call(body, in_specs, out_specs, out_shapes, args):
    res = pl.pallas_call(
        body,
        grid=(NBLK,),
        in_specs=in_specs,
        out_specs=out_specs,
        out_shape=out_shapes,
    )(*args)
    return res[0] if len(out_shapes) == 1 else res


def _sds(*shape):
    return jax.ShapeDtypeStruct(shape, jnp.float32)


@jax.jit
def kernel(x, edge_index, W1, b1, W2, b2, W3, b3):
    pad = PE - E
    src = jnp.concatenate([edge_index[0],
                           jnp.full((pad,), NP - 1, jnp.int32)]).reshape(NB, 128)
    dst = jnp.concatenate([edge_index[1],
                           jnp.full((pad,), NP - 1, jnp.int32)]).reshape(NB, 128)
    x_p = jnp.zeros((NP, IN_DIM), jnp.float32).at[:N].set(x)
    z16 = jnp.zeros((STRIPE, CH), jnp.float32)
    z1 = jnp.zeros((STRIPE,), jnp.float32)
    ones_t = jnp.ones((NP,), jnp.float32)

    degp = _sc_agg_scalar(ones_t, src, dst, z1)

    dinv, xs0, xs1, xs2 = _tc_call(
        _tc1_body,
        [_rowspec(1), _rowspec(1), _rowspec(IN_DIM)],
        [_rowspec(1)] + [_rowspec(CH)] * 3,
        [_sds(NP, 1)] + [_sds(NP, CH)] * 3,
        (degp[:NP].reshape(NP, 1), degp[NP:].reshape(NP, 1), x_p),
    )

    agg1 = _sc_agg_rows([xs0, xs1, xs2], src, dst, z16)

    tc2_out = _tc_call(
        _tc2_body,
        [_rowspec(CH)] * 6 + [_rowspec(IN_DIM), _rowspec(1),
                              _fullspec((IN_DIM, H1)), _fullspec((1, H1)),
                              _fullspec((H1, H2))],
        [_rowspec(H2)] + [_rowspec(CH)] * 7,
        [_sds(NP, H2)] + [_sds(NP, CH)] * 7,
        (agg1[0][:NP], agg1[0][NP:], agg1[1][:NP], agg1[1][NP:],
         agg1[2][:NP], agg1[2][NP:], x_p, dinv,
         W1, b1.reshape(1, H1), W2),
    )
    t2, xt = tc2_out[0], tc2_out[1:]

    agg2 = _sc_agg_rows(list(xt), src, dst, z16)

    tc3_in = []
    for a in agg2:
        tc3_in += [a[:NP], a[NP:]]
    ts3, t3 = _tc_call(
        _tc3_body,
        [_rowspec(CH)] * 14 + [_rowspec(H2), _rowspec(1),
                               _fullspec((1, H2)), _fullspec((H2, 1))],
        [_rowspec(1), _rowspec(1)],
        [_sds(NP, 1), _sds(NP, 1)],
        tuple(tc3_in) + (t2, dinv, b2.reshape(1, H2), W3),
    )

    agg3 = _sc_agg_scalar(ts3.reshape(NP), src, dst, z1)

    out = _tc_call(
        _tc4_body,
        [_rowspec(1), _rowspec(1), _rowspec(1), _rowspec(1), _fullspec((1, 1))],
        [_rowspec(1)],
        [_sds(NP, 1)],
        (agg3[:NP].reshape(NP, 1), agg3[NP:].reshape(NP, 1), t3, dinv,
         b3.reshape(1, 1)),
    )
    return out[:N, 0]


# speedup vs baseline: 14.4854x; 1.4003x over previous
import jax
import jax.numpy as jnp
from jax import lax
from jax.experimental import pallas as pl
from jax.experimental.pallas import tpu as pltpu
from jax.experimental.pallas import tpu_sc as plsc

N = 50000
E = 1_600_000
IN_DIM, H1, H2 = 48, 300, 100

NP = 50176
STRIPE = NP // 32
WBC = STRIPE // 8
PE = 1_638_400
NC, NS = 2, 16
PERW_E = PE // (NC * NS)

CH = 16
M = 8

NB = PE // 128
PERW = NB // (NC * NS)
GRP = 8
NGRP = PERW // GRP

BLK = 512
NBLK = NP // BLK


def _mesh():
    return plsc.VectorSubcoreMesh(core_axis_name="c", subcore_axis_name="s")


def _sc_agg_rows(tables, src2d, dst2d, zeros):
    nt = len(tables)

    def body(*refs):
        t_refs = refs[:nt]
        src_r, dst_r, z_r = refs[nt], refs[nt + 1], refs[nt + 2]
        out_refs = refs[nt + 3:nt + 3 + nt]
        sc = list(refs[nt + 3 + nt:])
        acc = sc[0]
        sv2, dv2 = sc[1], sc[2]
        rv = sc[3:3 + M]
        zv = sc[3 + M]
        si = sc[4 + M:6 + M]
        sg = sc[6 + M:6 + 2 * M]
        ss = sc[6 + 2 * M:6 + 3 * M]
        c = lax.axis_index("c")
        s = lax.axis_index("s")
        w = s * NC + c
        base_row = w * PERW

        for ti in range(nt):
            tbl = t_refs[ti]

            pltpu.sync_copy(z_r, zv)
            for q in range(STRIPE // WBC):
                pltpu.sync_copy(zv, acc.at[pl.ds(s * STRIPE + q * WBC, WBC)])
            plsc.subcore_barrier()

            def step(b, _):
                row0 = base_row + b * M
                h1 = pltpu.async_copy(src_r.at[pl.ds(row0, M)], sv2, si[0])
                h2 = pltpu.async_copy(dst_r.at[pl.ds(row0, M)], dv2, si[1])
                h1.wait()
                h2.wait()
                hg = []
                for j in range(M):
                    hg.append(pltpu.async_copy(tbl.at[sv2.at[j]], rv[j], sg[j]))
                hs = []
                for j in range(M):
                    hg[j].wait()
                    hs.append(pltpu.async_copy(rv[j], acc.at[dv2.at[j]], ss[j],
                                               add=True))
                for j in range(M):
                    hs[j].wait()
                return 0

            lax.fori_loop(0, PERW // M, step, 0)
            plsc.subcore_barrier()
            for q in range(STRIPE // WBC):
                pltpu.sync_copy(acc.at[pl.ds(s * STRIPE + q * WBC, WBC)], zv)
                pltpu.sync_copy(
                    zv,
                    out_refs[ti].at[pl.ds(c * NP + s * STRIPE + q * WBC, WBC)])
            plsc.subcore_barrier()

    out_t = [jax.ShapeDtypeStruct((2 * NP, CH), jnp.float32) for _ in range(nt)]
    scratch = (
        [pltpu.VMEM_SHARED((NP, CH), jnp.float32)]
        + [pltpu.VMEM((M, 128), jnp.int32) for _ in range(2)]
        + [pltpu.VMEM((128, CH), jnp.float32) for _ in range(M)]
        + [pltpu.VMEM((WBC, CH), jnp.float32)]
        + [pltpu.SemaphoreType.DMA for _ in range(2 + 2 * M)]
    )
    f = pl.kernel(body, out_type=out_t, mesh=_mesh(), scratch_types=scratch,
                  compiler_params=pltpu.CompilerParams(
                      use_tc_tiling_on_sc=False))
    outs = f(*tables, src2d, dst2d, zeros)
    return list(outs) if nt > 1 else [outs]


def _sc_agg_scalar(table, src2d, dst2d, zeros1):
    def body(t_r, src_r, dst_r, z_r, out_r, acc, tbl_v, src_v, dst_v, vals_v, zv):
        c = lax.axis_index("c")
        s = lax.axis_index("s")
        w = s * NC + c
        base_row = w * PERW
        pltpu.sync_copy(z_r, zv)
        pltpu.sync_copy(zv, acc.at[pl.ds(s * STRIPE, STRIPE)])
        pltpu.sync_copy(t_r, tbl_v)
        plsc.subcore_barrier()

        def grp(g, _):
            row0 = base_row + g * GRP
            pltpu.sync_copy(src_r.at[pl.ds(row0, GRP)], src_v)
            pltpu.sync_copy(dst_r.at[pl.ds(row0, GRP)], dst_v)
            for j in range(GRP):
                for k in range(8):
                    idx16 = src_v[j, pl.ds(k * 16, 16)]
                    vals_v[pl.ds(k * 16, 16)] = plsc.load_gather(tbl_v, [idx16])
                pltpu.sync_copy(vals_v, acc.at[dst_v.at[j]], add=True)
            return 0

        lax.fori_loop(0, NGRP, grp, 0)
        plsc.subcore_barrier()
        pltpu.sync_copy(acc.at[pl.ds(s * STRIPE, STRIPE)], zv)
        pltpu.sync_copy(zv, out_r.at[pl.ds(c * NP + s * STRIPE, STRIPE)])

    scratch = [
        pltpu.VMEM_SHARED((NP,), jnp.float32),
        pltpu.VMEM((NP,), jnp.float32),
        pltpu.VMEM((GRP, 128), jnp.int32),
        pltpu.VMEM((GRP, 128), jnp.int32),
        pltpu.VMEM((128,), jnp.float32),
        pltpu.VMEM((STRIPE,), jnp.float32),
    ]
    f = pl.kernel(body, out_type=jax.ShapeDtypeStruct((2 * NP,), jnp.float32),
                  mesh=_mesh(), scratch_types=scratch,
                  compiler_params=pltpu.CompilerParams(
                      needs_layout_passes=False, use_tc_tiling_on_sc=False))
    return f(table, src2d, dst2d, zeros1)


def _row_mask(pid):
    rows = pid * BLK + lax.broadcasted_iota(jnp.int32, (BLK, 1), 0)
    return (rows < N).astype(jnp.float32)


def _tc1_body(d0, d1, x, dinv_o, x0_o, x1_o, x2_o):
    deg = d0[...] + d1[...] + 1.0
    dinv = lax.rsqrt(deg)
    m = _row_mask(pl.program_id(0))
    xs = (m * dinv) * x[...]
    dinv_o[...] = dinv
    x0_o[...] = xs[:, 0:16]
    x1_o[...] = xs[:, 16:32]
    x2_o[...] = xs[:, 32:48]


def _tc2_body(a00, a01, a10, a11, a20, a21, x, dinv_r, W1, b1r, W2, *outs):
    agg = jnp.concatenate([a00[...] + a01[...], a10[...] + a11[...],
                           a20[...] + a21[...]], axis=1)
    dinv = dinv_r[...]
    pre = dinv * agg + (dinv * dinv) * x[...]
    h1 = jnp.maximum(jnp.dot(pre, W1[...],
                             preferred_element_type=jnp.float32) + b1r[...], 0.0)
    t2 = jnp.dot(h1, W2[...], preferred_element_type=jnp.float32)
    m = _row_mask(pl.program_id(0))
    xs2 = (m * dinv) * t2
    outs[0][...] = t2
    for ci in range(6):
        outs[1 + ci][...] = xs2[:, 16 * ci:16 * ci + 16]
    outs[7][...] = jnp.concatenate(
        [xs2[:, 96:100], jnp.zeros((BLK, 12), jnp.float32)], axis=1)


def _tc3_body(*refs):
    aggs = refs[:14]
    t2, dinv_r, b2r, W3, ts3_o, t3_o = refs[14:]
    agg = jnp.concatenate([aggs[2 * i][...] + aggs[2 * i + 1][...]
                           for i in range(7)], axis=1)[:, :100]
    dinv = dinv_r[...]
    h2 = jnp.maximum(dinv * agg + (dinv * dinv) * t2[...] + b2r[...], 0.0)
    t3 = jnp.dot(h2, W3[...], preferred_element_type=jnp.float32)
    m = _row_mask(pl.program_id(0))
    ts3_o[...] = (m * dinv) * t3
    t3_o[...] = t3


def _tc4_body(g0, g1, t3, dinv_r, b3r, out_o):
    dinv = dinv_r[...]
    out_o[...] = dinv * (g0[...] + g1[...]) + (dinv * dinv) * t3[...] + b3r[...]


def _rowspec(cols):
    return pl.BlockSpec((BLK, cols), lambda i: (i, 0))


def _fullspec(shape):
    return pl.BlockSpec(shape, lambda i: tuple(0 for _ in shape))


def _tc_call(body, in_specs, out_specs, out_shapes, args):
    res = pl.pallas_call(
        body,
        grid=(NBLK,),
        in_specs=in_specs,
        out_specs=out_specs,
        out_shape=out_shapes,
    )(*args)
    return res[0] if len(out_shapes) == 1 else res


def _sds(*shape):
    return jax.ShapeDtypeStruct(shape, jnp.float32)


@jax.jit
def kernel(x, edge_index, W1, b1, W2, b2, W3, b3):
    pad = PE - E
    src2 = jnp.concatenate([edge_index[0],
                            jnp.full((pad,), NP - 1, jnp.int32)]).reshape(NB, 128)
    dst2 = jnp.concatenate([edge_index[1],
                            jnp.full((pad,), NP - 1, jnp.int32)]).reshape(NB, 128)
    x_p = jnp.zeros((NP, IN_DIM), jnp.float32).at[:N].set(x)
    zch = jnp.zeros((WBC, CH), jnp.float32)
    z1 = jnp.zeros((STRIPE,), jnp.float32)
    ones_t = jnp.ones((NP,), jnp.float32)

    degp = _sc_agg_scalar(ones_t, src2, dst2, z1)

    dinv, xt0, xt1, xt2 = _tc_call(
        _tc1_body,
        [_rowspec(1), _rowspec(1), _rowspec(IN_DIM)],
        [_rowspec(1)] + [_rowspec(CH)] * 3,
        [_sds(NP, 1)] + [_sds(NP, CH)] * 3,
        (degp[:NP].reshape(NP, 1), degp[NP:].reshape(NP, 1), x_p),
    )

    agg1 = _sc_agg_rows([xt0, xt1, xt2], src2, dst2, zch)

    tc2_out = _tc_call(
        _tc2_body,
        [_rowspec(CH)] * 6 + [_rowspec(IN_DIM), _rowspec(1),
                              _fullspec((IN_DIM, H1)), _fullspec((1, H1)),
                              _fullspec((H1, H2))],
        [_rowspec(H2)] + [_rowspec(CH)] * 7,
        [_sds(NP, H2)] + [_sds(NP, CH)] * 7,
        (agg1[0][:NP], agg1[0][NP:], agg1[1][:NP], agg1[1][NP:],
         agg1[2][:NP], agg1[2][NP:], x_p, dinv,
         W1, b1.reshape(1, H1), W2),
    )
    t2, xt = tc2_out[0], tc2_out[1:]

    agg2 = _sc_agg_rows(list(xt), src2, dst2, zch)

    tc3_in = []
    for a in agg2:
        tc3_in += [a[:NP], a[NP:]]
    ts3, t3 = _tc_call(
        _tc3_body,
        [_rowspec(CH)] * 14 + [_rowspec(H2), _rowspec(1),
                               _fullspec((1, H2)), _fullspec((H2, 1))],
        [_rowspec(1), _rowspec(1)],
        [_sds(NP, 1), _sds(NP, 1)],
        tuple(tc3_in) + (t2, dinv, b2.reshape(1, H2), W3),
    )

    agg3 = _sc_agg_scalar(ts3.reshape(NP), src2, dst2, z1)

    out = _tc_call(
        _tc4_body,
        [_rowspec(1), _rowspec(1), _rowspec(1), _rowspec(1), _fullspec((1, 1))],
        [_rowspec(1)],
        [_sds(NP, 1)],
        (agg3[:NP].reshape(NP, 1), agg3[NP:].reshape(NP, 1), t3, dinv,
         b3.reshape(1, 1)),
    )
    return out[:N, 0]


# speedup vs baseline: 16.9243x; 1.1684x over previous
import jax
import jax.numpy as jnp
from jax import lax
from jax.experimental import pallas as pl
from jax.experimental.pallas import tpu as pltpu
from jax.experimental.pallas import tpu_sc as plsc

N = 50000
E = 1_600_000
IN_DIM, H1, H2 = 48, 300, 100

NP = 50176
STRIPE = NP // 32
WBC = STRIPE // 8
PE = 1_638_400
NC, NS = 2, 16
PERW_E = PE // (NC * NS)

CH = 16
M = 8
ASPLIT = 576

NB = PE // 128
PERW = NB // (NC * NS)
GRP = 8
NGRP = PERW // GRP

BLK = 512
NBLK = NP // BLK


def _mesh():
    return plsc.VectorSubcoreMesh(core_axis_name="c", subcore_axis_name="s")


def _sc_agg_rows(tables, src2d, dst2d, zeros):
    nt = len(tables)

    def body(*refs):
        t_refs = refs[:nt]
        src_r, dst_r, z_r = refs[nt], refs[nt + 1], refs[nt + 2]
        out_refs = refs[nt + 3:nt + 3 + nt]
        sc = list(refs[nt + 3 + nt:])
        acc = sc[0]
        sv2, dv2 = sc[1], sc[2]
        rv = sc[3:3 + M]
        zv = sc[3 + M]
        si = sc[4 + M:6 + M]
        sg = sc[6 + M:6 + 2 * M]
        ss = sc[6 + 2 * M:6 + 3 * M]
        c = lax.axis_index("c")
        s = lax.axis_index("s")
        base_row = s * (2 * PERW) + c * ASPLIT
        nit = lax.select(c == 0, ASPLIT // M, (2 * PERW - ASPLIT) // M)

        for ti in range(nt):
            tbl = t_refs[ti]

            pltpu.sync_copy(z_r, zv)
            for q in range(STRIPE // WBC):
                pltpu.sync_copy(zv, acc.at[pl.ds(s * STRIPE + q * WBC, WBC)])
            plsc.subcore_barrier()

            def step(b, _):
                row0 = base_row + b * M
                h1 = pltpu.async_copy(src_r.at[pl.ds(row0, M)], sv2, si[0])
                h2 = pltpu.async_copy(dst_r.at[pl.ds(row0, M)], dv2, si[1])
                h1.wait()
                h2.wait()
                hg = []
                for j in range(M):
                    hg.append(pltpu.async_copy(tbl.at[sv2.at[j]], rv[j], sg[j]))
                hs = []
                for j in range(M):
                    hg[j].wait()
                    hs.append(pltpu.async_copy(rv[j], acc.at[dv2.at[j]], ss[j],
                                               add=True))
                for j in range(M):
                    hs[j].wait()
                return 0

            lax.fori_loop(0, nit, step, 0)
            plsc.subcore_barrier()
            for q in range(STRIPE // WBC):
                pltpu.sync_copy(acc.at[pl.ds(s * STRIPE + q * WBC, WBC)], zv)
                pltpu.sync_copy(
                    zv,
                    out_refs[ti].at[pl.ds(c * NP + s * STRIPE + q * WBC, WBC)])
            plsc.subcore_barrier()

    out_t = [jax.ShapeDtypeStruct((2 * NP, CH), jnp.float32) for _ in range(nt)]
    scratch = (
        [pltpu.VMEM_SHARED((NP, CH), jnp.float32)]
        + [pltpu.VMEM((M, 128), jnp.int32) for _ in range(2)]
        + [pltpu.VMEM((128, CH), jnp.float32) for _ in range(M)]
        + [pltpu.VMEM((WBC, CH), jnp.float32)]
        + [pltpu.SemaphoreType.DMA for _ in range(2 + 2 * M)]
    )
    f = pl.kernel(body, out_type=out_t, mesh=_mesh(), scratch_types=scratch,
                  compiler_params=pltpu.CompilerParams(
                      use_tc_tiling_on_sc=False))
    outs = f(*tables, src2d, dst2d, zeros)
    return list(outs) if nt > 1 else [outs]


def _sc_agg_scalar(table, src2d, dst2d, zeros1):
    def body(t_r, src_r, dst_r, z_r, out_r, acc, tbl_v, src_v, dst_v, vals_v, zv):
        c = lax.axis_index("c")
        s = lax.axis_index("s")
        w = s * NC + c
        base_row = w * PERW
        pltpu.sync_copy(z_r, zv)
        pltpu.sync_copy(zv, acc.at[pl.ds(s * STRIPE, STRIPE)])
        pltpu.sync_copy(t_r, tbl_v)
        plsc.subcore_barrier()

        def grp(g, _):
            row0 = base_row + g * GRP
            pltpu.sync_copy(src_r.at[pl.ds(row0, GRP)], src_v)
            pltpu.sync_copy(dst_r.at[pl.ds(row0, GRP)], dst_v)
            for j in range(GRP):
                for k in range(8):
                    idx16 = src_v[j, pl.ds(k * 16, 16)]
                    vals_v[pl.ds(k * 16, 16)] = plsc.load_gather(tbl_v, [idx16])
                pltpu.sync_copy(vals_v, acc.at[dst_v.at[j]], add=True)
            return 0

        lax.fori_loop(0, NGRP, grp, 0)
        plsc.subcore_barrier()
        pltpu.sync_copy(acc.at[pl.ds(s * STRIPE, STRIPE)], zv)
        pltpu.sync_copy(zv, out_r.at[pl.ds(c * NP + s * STRIPE, STRIPE)])

    scratch = [
        pltpu.VMEM_SHARED((NP,), jnp.float32),
        pltpu.VMEM((NP,), jnp.float32),
        pltpu.VMEM((GRP, 128), jnp.int32),
        pltpu.VMEM((GRP, 128), jnp.int32),
        pltpu.VMEM((128,), jnp.float32),
        pltpu.VMEM((STRIPE,), jnp.float32),
    ]
    f = pl.kernel(body, out_type=jax.ShapeDtypeStruct((2 * NP,), jnp.float32),
                  mesh=_mesh(), scratch_types=scratch,
                  compiler_params=pltpu.CompilerParams(
                      needs_layout_passes=False, use_tc_tiling_on_sc=False))
    return f(table, src2d, dst2d, zeros1)


def _row_mask(pid):
    rows = pid * BLK + lax.broadcasted_iota(jnp.int32, (BLK, 1), 0)
    return (rows < N).astype(jnp.float32)


def _tc1_body(d0, d1, x, dinv_o, x0_o, x1_o, x2_o):
    deg = d0[...] + d1[...] + 1.0
    dinv = lax.rsqrt(deg)
    m = _row_mask(pl.program_id(0))
    xs = (m * dinv) * x[...]
    dinv_o[...] = dinv
    x0_o[...] = xs[:, 0:16]
    x1_o[...] = xs[:, 16:32]
    x2_o[...] = xs[:, 32:48]


def _tc2_body(a00, a01, a10, a11, a20, a21, x, dinv_r, W1, b1r, W2, *outs):
    agg = jnp.concatenate([a00[...] + a01[...], a10[...] + a11[...],
                           a20[...] + a21[...]], axis=1)
    dinv = dinv_r[...]
    pre = dinv * agg + (dinv * dinv) * x[...]
    h1 = jnp.maximum(jnp.dot(pre, W1[...],
                             preferred_element_type=jnp.float32) + b1r[...], 0.0)
    t2 = jnp.dot(h1, W2[...], preferred_element_type=jnp.float32)
    m = _row_mask(pl.program_id(0))
    xs2 = (m * dinv) * t2
    outs[0][...] = t2
    for ci in range(6):
        outs[1 + ci][...] = xs2[:, 16 * ci:16 * ci + 16]
    outs[7][...] = jnp.concatenate(
        [xs2[:, 96:100], jnp.zeros((BLK, 12), jnp.float32)], axis=1)


def _tc3_body(*refs):
    aggs = refs[:14]
    t2, dinv_r, b2r, W3, ts3_o, t3_o = refs[14:]
    agg = jnp.concatenate([aggs[2 * i][...] + aggs[2 * i + 1][...]
                           for i in range(7)], axis=1)[:, :100]
    dinv = dinv_r[...]
    h2 = jnp.maximum(dinv * agg + (dinv * dinv) * t2[...] + b2r[...], 0.0)
    t3 = jnp.dot(h2, W3[...], preferred_element_type=jnp.float32)
    m = _row_mask(pl.program_id(0))
    ts3_o[...] = (m * dinv) * t3
    t3_o[...] = t3


def _tc4_body(g0, g1, t3, dinv_r, b3r, out_o):
    dinv = dinv_r[...]
    out_o[...] = dinv * (g0[...] + g1[...]) + (dinv * dinv) * t3[...] + b3r[...]


def _rowspec(cols):
    return pl.BlockSpec((BLK, cols), lambda i: (i, 0))


def _fullspec(shape):
    return pl.BlockSpec(shape, lambda i: tuple(0 for _ in shape))


def _tc_call(body, in_specs, out_specs, out_shapes, args):
    res = pl.pallas_call(
        body,
        grid=(NBLK,),
        in_specs=in_specs,
        out_specs=out_specs,
        out_shape=out_shapes,
    )(*args)
    return res[0] if len(out_shapes) == 1 else res


def _sds(*shape):
    return jax.ShapeDtypeStruct(shape, jnp.float32)


@jax.jit
def kernel(x, edge_index, W1, b1, W2, b2, W3, b3):
    pad = PE - E
    src2 = jnp.concatenate([edge_index[0],
                            jnp.full((pad,), NP - 1, jnp.int32)]).reshape(NB, 128)
    dst2 = jnp.concatenate([edge_index[1],
                            jnp.full((pad,), NP - 1, jnp.int32)]).reshape(NB, 128)
    x_p = jnp.zeros((NP, IN_DIM), jnp.float32).at[:N].set(x)
    zch = jnp.zeros((WBC, CH), jnp.float32)
    z1 = jnp.zeros((STRIPE,), jnp.float32)
    ones_t = jnp.ones((NP,), jnp.float32)

    degp = _sc_agg_scalar(ones_t, src2, dst2, z1)

    dinv, xt0, xt1, xt2 = _tc_call(
        _tc1_body,
        [_rowspec(1), _rowspec(1), _rowspec(IN_DIM)],
        [_rowspec(1)] + [_rowspec(CH)] * 3,
        [_sds(NP, 1)] + [_sds(NP, CH)] * 3,
        (degp[:NP].reshape(NP, 1), degp[NP:].reshape(NP, 1), x_p),
    )

    agg1 = _sc_agg_rows([xt0, xt1, xt2], src2, dst2, zch)

    tc2_out = _tc_call(
        _tc2_body,
        [_rowspec(CH)] * 6 + [_rowspec(IN_DIM), _rowspec(1),
                              _fullspec((IN_DIM, H1)), _fullspec((1, H1)),
                              _fullspec((H1, H2))],
        [_rowspec(H2)] + [_rowspec(CH)] * 7,
        [_sds(NP, H2)] + [_sds(NP, CH)] * 7,
        (agg1[0][:NP], agg1[0][NP:], agg1[1][:NP], agg1[1][NP:],
         agg1[2][:NP], agg1[2][NP:], x_p, dinv,
         W1, b1.reshape(1, H1), W2),
    )
    t2, xt = tc2_out[0], tc2_out[1:]

    agg2 = _sc_agg_rows(list(xt), src2, dst2, zch)

    tc3_in = []
    for a in agg2:
        tc3_in += [a[:NP], a[NP:]]
    ts3, t3 = _tc_call(
        _tc3_body,
        [_rowspec(CH)] * 14 + [_rowspec(H2), _rowspec(1),
                               _fullspec((1, H2)), _fullspec((H2, 1))],
        [_rowspec(1), _rowspec(1)],
        [_sds(NP, 1), _sds(NP, 1)],
        tuple(tc3_in) + (t2, dinv, b2.reshape(1, H2), W3),
    )

    agg3 = _sc_agg_scalar(ts3.reshape(NP), src2, dst2, z1)

    out = _tc_call(
        _tc4_body,
        [_rowspec(1), _rowspec(1), _rowspec(1), _rowspec(1), _fullspec((1, 1))],
        [_rowspec(1)],
        [_sds(NP, 1)],
        (agg3[:NP].reshape(NP, 1), agg3[NP:].reshape(NP, 1), t3, dinv,
         b3.reshape(1, 1)),
    )
    return out[:N, 0]
